# Initial kernel scaffold; baseline (speedup 1.0000x reference)
import jax
import jax.numpy as jnp
from jax.experimental import pallas as pl


def kernel(U, rest, edges, cand_v, cand_e):
    raise NotImplementedError("write your pallas kernel here")


# speedup vs baseline: 9.5037x; 9.5037x over previous
import functools

import jax
import jax.numpy as jnp
from jax import lax
from jax.experimental import pallas as pl
from jax.experimental.pallas import tpu as pltpu
from jax.experimental.pallas import tpu_sc as plsc

DHAT = 0.05
DHAT2 = DHAT * DHAT
INV_DHAT2 = 1.0 / DHAT2

N_CORES = 2
N_SUBCORES = 16
N_TILES = N_CORES * N_SUBCORES
LANES = 16

CAND_PER_TILE = 6400
HALF = CAND_PER_TILE // 2
GATHER_W = 128
N_GATHERS = HALF // GATHER_W
N_CHUNKS = HALF // LANES

CURR_SLICE = 6256
CURR_PAD = N_SUBCORES * CURR_SLICE


def _ln(x):
    xi = plsc.bitcast(x, jnp.int32)
    e = ((xi >> 23) & 0xFF) - 127
    m = plsc.bitcast((xi & 0x7FFFFF) | 0x3F800000, jnp.float32)
    big = m > 1.4142135
    m = jnp.where(big, m * 0.5, m)
    ef = e.astype(jnp.float32) + jnp.where(big, 1.0, 0.0)
    t = (m - 1.0) / (m + 1.0)
    t2 = t * t
    p = t * (2.0 + t2 * (0.6666667 + t2 * (0.4 + t2 * (0.2857143 + t2 * 0.22222223))))
    return ef * 0.6931472 + p


def _make_sc_kernel(n_cands):
    mesh = plsc.VectorSubcoreMesh(core_axis_name="c", subcore_axis_name="s")

    @functools.partial(
        pl.kernel,
        out_type=jax.ShapeDtypeStruct((N_TILES, LANES), jnp.float32),
        mesh=mesh,
        scratch_types=[
            pltpu.VMEM((CURR_PAD,), jnp.float32),
            pltpu.VMEM((CURR_SLICE,), jnp.float32),
            pltpu.VMEM((HALF,), jnp.int32),
            pltpu.VMEM((N_GATHERS, GATHER_W), jnp.int32),
            pltpu.VMEM((HALF,), jnp.int32),
            pltpu.VMEM((LANES,), jnp.float32),
            pltpu.VMEM_SHARED((CURR_PAD,), jnp.float32),
            pltpu.SemaphoreType.DMA,
        ],
        compiler_params=pltpu.CompilerParams(
            needs_layout_passes=False, use_tc_tiling_on_sc=False),
    )
    def sck(rest_hbm, u_hbm, epk_hbm, cv_hbm, ce_hbm, out_hbm,
            curr_v, ubuf_v, cv_v, ce_v, epk_v, acc_v, curr_sh, sem):
        c = lax.axis_index("c")
        s = lax.axis_index("s")
        wid = s * N_CORES + c

        myoff = s * CURR_SLICE
        pltpu.sync_copy(rest_hbm.at[pl.ds(myoff, CURR_SLICE)],
                        curr_v.at[pl.ds(myoff, CURR_SLICE)])
        pltpu.sync_copy(u_hbm.at[pl.ds(myoff, CURR_SLICE)], ubuf_v)

        @pl.loop(0, CURR_SLICE // LANES)
        def _(j):
            d = pl.ds(myoff + j * LANES, LANES)
            curr_v[d] = curr_v[d] + ubuf_v[pl.ds(j * LANES, LANES)]

        pltpu.sync_copy(curr_v.at[pl.ds(myoff, CURR_SLICE)],
                        curr_sh.at[pl.ds(myoff, CURR_SLICE)])
        plsc.subcore_barrier()
        pltpu.sync_copy(curr_sh, curr_v)

        acc_v[...] = jnp.zeros((LANES,), jnp.float32)
        lanes = lax.iota(jnp.int32, LANES)

        for h in range(2):
            widh = wid * 2 + h
            pltpu.sync_copy(cv_hbm.at[widh], cv_v)
            pltpu.sync_copy(ce_hbm.at[widh], ce_v)

            @pl.loop(0, N_GATHERS)
            def _(k):
                pltpu.make_async_copy(
                    epk_hbm.at[ce_v.at[k]],
                    epk_v.at[pl.ds(k * GATHER_W, GATHER_W)],
                    sem).start()

            @pl.loop(0, N_GATHERS)
            def _(k):
                pltpu.make_async_copy(
                    epk_hbm.at[ce_v.at[k]],
                    epk_v.at[pl.ds(k * GATHER_W, GATHER_W)],
                    sem).wait()

            base_g = widh * HALF

            @pl.loop(0, N_CHUNKS)
            def _(j):
                jb = j * LANES
                cv = cv_v[pl.ds(jb, LANES)]
                pk = epk_v[pl.ds(jb, LANES)]
                e0i = pk & 0xFFFF
                e1i = (pk >> 16) & 0xFFFF
                px = plsc.load_gather(curr_v, [cv * 2])
                py = plsc.load_gather(curr_v, [cv * 2 + 1])
                e0x = plsc.load_gather(curr_v, [e0i * 2])
                e0y = plsc.load_gather(curr_v, [e0i * 2 + 1])
                e1x = plsc.load_gather(curr_v, [e1i * 2])
                e1y = plsc.load_gather(curr_v, [e1i * 2 + 1])

                dex = e1x - e0x
                dey = e1y - e0y
                dd = jnp.maximum(dex * dex + dey * dey, 1e-12)
                qx = px - e0x
                qy = py - e0y
                t = (qx * dex + qy * dey) / dd
                t = jnp.minimum(jnp.maximum(t, 0.0), 1.0)
                cx = e0x + t * dex
                cy = e0y + t * dey
                dx = px - cx
                dy = py - cy
                d2 = dx * dx + dy * dy

                active = (d2 < DHAT2) & (d2 > 0.0)
                d2s = jnp.where(active, d2, DHAT2)
                diff = d2s - DHAT2
                b = -(diff * diff) * _ln(d2s * INV_DHAT2)
                g = base_g + jb + lanes
                b = jnp.where(active & (g < n_cands), b, 0.0)
                acc_v[...] = acc_v[...] + b

        pltpu.sync_copy(acc_v, out_hbm.at[wid])

    return sck


def kernel(U, rest, edges, cand_v, cand_e):
    n_verts = rest.shape[0]
    n_cands = cand_v.shape[0]
    flat = 2 * n_verts
    rest_p = jnp.pad(rest.reshape(-1), (0, CURR_PAD - flat))
    u_p = jnp.pad(U.reshape(-1), (0, CURR_PAD - flat))
    epk = edges[:, 0] | (edges[:, 1] << 16)
    pad_c = N_TILES * CAND_PER_TILE - n_cands
    cv = jnp.pad(cand_v, (0, pad_c)).reshape(N_TILES * 2, HALF)
    ce = jnp.pad(cand_e, (0, pad_c)).reshape(N_TILES * 2, N_GATHERS, GATHER_W)
    out = _make_sc_kernel(n_cands)(rest_p, u_p, epk, cv, ce)
    return jnp.sum(out)


# speedup vs baseline: 11.2309x; 1.1817x over previous
import functools

import jax
import jax.numpy as jnp
from jax import lax
from jax.experimental import pallas as pl
from jax.experimental.pallas import tpu as pltpu
from jax.experimental.pallas import tpu_sc as plsc

DHAT = 0.05
DHAT2 = DHAT * DHAT
INV_DHAT2 = 1.0 / DHAT2

N_CORES = 2
N_SUBCORES = 16
N_TILES = N_CORES * N_SUBCORES
LANES = 16

CAND_PER_TILE = 6400
GATHER_W = 128
N_GATHERS = CAND_PER_TILE // GATHER_W
N_CHUNKS = CAND_PER_TILE // LANES

CURR_SLICE = 6256
CURR_PAD = N_SUBCORES * CURR_SLICE


def _ln(x):
    xi = plsc.bitcast(x, jnp.int32)
    e = ((xi >> 23) & 0xFF) - 127
    m = plsc.bitcast((xi & 0x7FFFFF) | 0x3F800000, jnp.float32)
    big = m > 1.4142135
    m = jnp.where(big, m * 0.5, m)
    ef = e.astype(jnp.float32) + jnp.where(big, 1.0, 0.0)
    t = (m - 1.0) / (m + 1.0)
    t2 = t * t
    p = t * (2.0 + t2 * (0.6666667 + t2 * (0.4 + t2 * (0.2857143 + t2 * 0.22222223))))
    return ef * 0.6931472 + p


def _make_sc_kernel(n_cands):
    mesh = plsc.VectorSubcoreMesh(core_axis_name="c", subcore_axis_name="s")

    @functools.partial(
        pl.kernel,
        out_type=jax.ShapeDtypeStruct((N_TILES, LANES), jnp.float32),
        mesh=mesh,
        scratch_types=[
            pltpu.VMEM((CURR_PAD,), jnp.float32),
            pltpu.VMEM((CURR_SLICE,), jnp.float32),
            pltpu.VMEM((CAND_PER_TILE // 2,), jnp.int32),
            pltpu.VMEM((N_GATHERS, GATHER_W), jnp.int32),
            pltpu.VMEM((CAND_PER_TILE,), jnp.int32),
            pltpu.VMEM((LANES,), jnp.float32),
            pltpu.VMEM_SHARED((CURR_PAD,), jnp.float32),
            pltpu.SemaphoreType.DMA,
        ],
        compiler_params=pltpu.CompilerParams(
            needs_layout_passes=False, use_tc_tiling_on_sc=False),
    )
    def sck(rest_hbm, u_hbm, epk_hbm, cv_hbm, ce_hbm, out_hbm,
            curr_v, ubuf_v, cv_v, ce_v, epk_v, acc_v, curr_sh, sem):
        c = lax.axis_index("c")
        s = lax.axis_index("s")
        wid = s * N_CORES + c

        pltpu.sync_copy(cv_hbm.at[wid], cv_v)
        pltpu.sync_copy(ce_hbm.at[wid], ce_v)

        @pl.loop(0, N_GATHERS)
        def _(k):
            pltpu.make_async_copy(
                epk_hbm.at[ce_v.at[k]],
                epk_v.at[pl.ds(k * GATHER_W, GATHER_W)],
                sem).start()

        myoff = s * CURR_SLICE
        pltpu.sync_copy(rest_hbm.at[pl.ds(myoff, CURR_SLICE)],
                        curr_v.at[pl.ds(myoff, CURR_SLICE)])
        pltpu.sync_copy(u_hbm.at[pl.ds(myoff, CURR_SLICE)], ubuf_v)

        @pl.loop(0, CURR_SLICE // LANES)
        def _(j):
            d = pl.ds(myoff + j * LANES, LANES)
            curr_v[d] = curr_v[d] + ubuf_v[pl.ds(j * LANES, LANES)]

        pltpu.sync_copy(curr_v.at[pl.ds(myoff, CURR_SLICE)],
                        curr_sh.at[pl.ds(myoff, CURR_SLICE)])
        plsc.subcore_barrier()
        pltpu.sync_copy(curr_sh, curr_v)

        @pl.loop(0, N_GATHERS)
        def _(k):
            pltpu.make_async_copy(
                epk_hbm.at[ce_v.at[k]],
                epk_v.at[pl.ds(k * GATHER_W, GATHER_W)],
                sem).wait()

        lanes = lax.iota(jnp.int32, LANES)
        base_g = wid * CAND_PER_TILE
        half = CAND_PER_TILE // 2

        def contrib(cv, pk, g):
            e0i = pk & 0xFFFF
            e1i = (pk >> 16) & 0xFFFF
            px = plsc.load_gather(curr_v, [cv * 2])
            py = plsc.load_gather(curr_v, [cv * 2 + 1])
            e0x = plsc.load_gather(curr_v, [e0i * 2])
            e0y = plsc.load_gather(curr_v, [e0i * 2 + 1])
            e1x = plsc.load_gather(curr_v, [e1i * 2])
            e1y = plsc.load_gather(curr_v, [e1i * 2 + 1])

            dex = e1x - e0x
            dey = e1y - e0y
            dd = jnp.maximum(dex * dex + dey * dey, 1e-12)
            qx = px - e0x
            qy = py - e0y
            t = (qx * dex + qy * dey) / dd
            t = jnp.minimum(jnp.maximum(t, 0.0), 1.0)
            cx = e0x + t * dex
            cy = e0y + t * dey
            dx = px - cx
            dy = py - cy
            d2 = dx * dx + dy * dy

            active = (d2 < DHAT2) & (d2 > 0.0)
            d2s = jnp.where(active, d2, DHAT2)
            diff = d2s - DHAT2
            b = -(diff * diff) * _ln(d2s * INV_DHAT2)
            return jnp.where(active & (g < n_cands), b, 0.0)

        @plsc.parallel_loop(0, N_CHUNKS // 2,
                            carry=jnp.zeros((LANES,), jnp.float32))
        def acc(j, acc_in):
            jb = j * LANES
            cvp = cv_v[pl.ds(jb, LANES)]
            pk_lo = epk_v[pl.ds(jb, LANES)]
            pk_hi = epk_v[pl.ds(half + jb, LANES)]
            b_lo = contrib(cvp & 0xFFFF, pk_lo, base_g + jb + lanes)
            b_hi = contrib((cvp >> 16) & 0xFFFF, pk_hi,
                           base_g + half + jb + lanes)
            return acc_in + b_lo + b_hi

        acc_v[...] = acc
        pltpu.sync_copy(acc_v, out_hbm.at[wid])

    return sck


def kernel(U, rest, edges, cand_v, cand_e):
    n_verts = rest.shape[0]
    n_cands = cand_v.shape[0]
    flat = 2 * n_verts
    rest_p = jnp.pad(rest.reshape(-1), (0, CURR_PAD - flat))
    u_p = jnp.pad(U.reshape(-1), (0, CURR_PAD - flat))
    epk = edges[:, 0] | (edges[:, 1] << 16)
    pad_c = N_TILES * CAND_PER_TILE - n_cands
    cv2 = jnp.pad(cand_v, (0, pad_c)).reshape(N_TILES, 2, CAND_PER_TILE // 2)
    cv = cv2[:, 0] | (cv2[:, 1] << 16)
    ce = jnp.pad(cand_e, (0, pad_c)).reshape(N_TILES, N_GATHERS, GATHER_W)
    out = _make_sc_kernel(n_cands)(rest_p, u_p, epk, cv, ce)
    return jnp.sum(out)


# speedup vs baseline: 19.5729x; 1.7428x over previous
import functools

import jax
import jax.numpy as jnp
from jax import lax
from jax.experimental import pallas as pl
from jax.experimental.pallas import tpu as pltpu
from jax.experimental.pallas import tpu_sc as plsc

DHAT = 0.05
DHAT2 = DHAT * DHAT
INV_DHAT2 = 1.0 / DHAT2

N_CORES = 2
N_SUBCORES = 16
N_TILES = N_CORES * N_SUBCORES
LANES = 16

CAND_PER_TILE = 6400
GATHER_W = 128
N_GATHERS = CAND_PER_TILE // GATHER_W
N_ITERS = CAND_PER_TILE // (2 * LANES)

TAB_SLICE = 3136
TAB_PAD = N_SUBCORES * TAB_SLICE


def _ln(x):
    xi = plsc.bitcast(x, jnp.int32)
    e = ((xi >> 23) & 0xFF) - 127
    m = plsc.bitcast((xi & 0x7FFFFF) | 0x3F800000, jnp.float32)
    big = m > 1.4142135
    m = jnp.where(big, m * 0.5, m)
    ef = e.astype(jnp.float32) + jnp.where(big, 1.0, 0.0)
    t = (m - 1.0) / (m + 1.0)
    t2 = t * t
    p = t * (2.0 + t2 * (0.6666667 + t2 * (0.4 + t2 * (0.2857143 + t2 * 0.22222223))))
    return ef * 0.6931472 + p


def _make_sc_kernel(n_cands):
    mesh = plsc.VectorSubcoreMesh(core_axis_name="c", subcore_axis_name="s")

    @functools.partial(
        pl.kernel,
        out_type=jax.ShapeDtypeStruct((N_TILES, LANES), jnp.float32),
        mesh=mesh,
        scratch_types=[
            pltpu.VMEM((TAB_PAD,), jnp.float32),
            pltpu.VMEM((TAB_PAD,), jnp.float32),
            pltpu.VMEM((TAB_SLICE,), jnp.float32),
            pltpu.VMEM((CAND_PER_TILE // 2,), jnp.int32),
            pltpu.VMEM((N_GATHERS, GATHER_W), jnp.int32),
            pltpu.VMEM((CAND_PER_TILE,), jnp.int32),
            pltpu.VMEM((LANES,), jnp.float32),
            pltpu.VMEM_SHARED((TAB_PAD,), jnp.float32),
            pltpu.VMEM_SHARED((TAB_PAD,), jnp.float32),
            pltpu.SemaphoreType.DMA,
        ],
        compiler_params=pltpu.CompilerParams(
            needs_layout_passes=False, use_tc_tiling_on_sc=False),
    )
    def sck(restx_hbm, resty_hbm, ux_hbm, uy_hbm, epk_hbm, cv_hbm, ce_hbm,
            out_hbm, currx_v, curry_v, ubuf_v, cv_v, ce_v, epk_v, acc_v,
            currx_sh, curry_sh, sem):
        c = lax.axis_index("c")
        s = lax.axis_index("s")
        wid = s * N_CORES + c

        pltpu.sync_copy(cv_hbm.at[wid], cv_v)
        pltpu.sync_copy(ce_hbm.at[wid], ce_v)

        @pl.loop(0, N_GATHERS)
        def _(k):
            pltpu.make_async_copy(
                epk_hbm.at[ce_v.at[k]],
                epk_v.at[pl.ds(k * GATHER_W, GATHER_W)],
                sem).start()

        myoff = s * TAB_SLICE
        sl = pl.ds(myoff, TAB_SLICE)
        for rtab, utab, tab_v, tab_sh in (
                (restx_hbm, ux_hbm, currx_v, currx_sh),
                (resty_hbm, uy_hbm, curry_v, curry_sh)):
            pltpu.sync_copy(rtab.at[sl], tab_v.at[sl])
            pltpu.sync_copy(utab.at[sl], ubuf_v)

            @pl.loop(0, TAB_SLICE // LANES)
            def _(j, tab_v=tab_v):
                d = pl.ds(myoff + j * LANES, LANES)
                tab_v[d] = tab_v[d] + ubuf_v[pl.ds(j * LANES, LANES)]

            pltpu.sync_copy(tab_v.at[sl], tab_sh.at[sl])

        plsc.subcore_barrier()
        pltpu.sync_copy(currx_sh, currx_v)
        pltpu.sync_copy(curry_sh, curry_v)

        @pl.loop(0, N_GATHERS)
        def _(k):
            pltpu.make_async_copy(
                epk_hbm.at[ce_v.at[k]],
                epk_v.at[pl.ds(k * GATHER_W, GATHER_W)],
                sem).wait()

        lanes = lax.iota(jnp.int32, LANES)
        base_g = wid * CAND_PER_TILE
        half = CAND_PER_TILE // 2

        def contrib(cv, pk, g):
            e0i = pk & 0xFFFF
            e1i = (pk >> 16) & 0xFFFF
            px = plsc.load_gather(currx_v, [cv])
            py = plsc.load_gather(curry_v, [cv])
            e0x = plsc.load_gather(currx_v, [e0i])
            e0y = plsc.load_gather(curry_v, [e0i])
            e1x = plsc.load_gather(currx_v, [e1i])
            e1y = plsc.load_gather(curry_v, [e1i])

            dex = e1x - e0x
            dey = e1y - e0y
            dd = jnp.maximum(dex * dex + dey * dey, 1e-12)
            qx = px - e0x
            qy = py - e0y
            t = (qx * dex + qy * dey) / dd
            t = jnp.minimum(jnp.maximum(t, 0.0), 1.0)
            cx = e0x + t * dex
            cy = e0y + t * dey
            dx = px - cx
            dy = py - cy
            d2 = dx * dx + dy * dy

            active = (d2 < DHAT2) & (d2 > 0.0)
            d2s = jnp.where(active, d2, DHAT2)
            diff = d2s - DHAT2
            b = -(diff * diff) * _ln(d2s * INV_DHAT2)
            return jnp.where(active & (g < n_cands), b, 0.0)

        @plsc.parallel_loop(0, N_ITERS, carry=jnp.zeros((LANES,), jnp.float32))
        def acc(j, acc_in):
            jb = j * LANES
            cvp = cv_v[pl.ds(jb, LANES)]
            pk_lo = epk_v[pl.ds(jb, LANES)]
            pk_hi = epk_v[pl.ds(half + jb, LANES)]
            b_lo = contrib(cvp & 0xFFFF, pk_lo, base_g + jb + lanes)
            b_hi = contrib((cvp >> 16) & 0xFFFF, pk_hi,
                           base_g + half + jb + lanes)
            return acc_in + b_lo + b_hi

        acc_v[...] = acc
        pltpu.sync_copy(acc_v, out_hbm.at[wid])

    return sck


def kernel(U, rest, edges, cand_v, cand_e):
    n_verts = rest.shape[0]
    n_cands = cand_v.shape[0]
    pad_v = TAB_PAD - n_verts
    restx = jnp.pad(rest[:, 0], (0, pad_v))
    resty = jnp.pad(rest[:, 1], (0, pad_v))
    ux = jnp.pad(U[:, 0], (0, pad_v))
    uy = jnp.pad(U[:, 1], (0, pad_v))
    epk = edges[:, 0] | (edges[:, 1] << 16)
    pad_c = N_TILES * CAND_PER_TILE - n_cands
    cv2 = jnp.pad(cand_v, (0, pad_c)).reshape(N_TILES, 2, CAND_PER_TILE // 2)
    cv = cv2[:, 0] | (cv2[:, 1] << 16)
    ce = jnp.pad(cand_e, (0, pad_c)).reshape(N_TILES, N_GATHERS, GATHER_W)
    out = _make_sc_kernel(n_cands)(restx, resty, ux, uy, epk, cv, ce)
    return jnp.sum(out)
